# baseline (speedup 1.0000x reference)
import numpy as np
import jax
import jax.numpy as jnp
from jax.experimental import pallas as pl
from jax.experimental.pallas import tpu as pltpu

_SUPPORT_P = 0.01
_SUPPORT_K = float(np.sqrt(-np.log(_SUPPORT_P)))
_BIG = 1e30
_THRESH = 1e20


def _bump_body(x_ref, minb_ref, maxb_ref, ctr_ref, ibw_ref, mag_ref,
               y_ref, any_ref, v0_ref):
    pid = pl.program_id(0)
    xb = x_ref[...]
    xt = jnp.transpose(xb, (0, 2, 1))
    minb = minb_ref[...]
    maxb = maxb_ref[...]
    ctr = ctr_ref[...]
    ibw = ibw_ref[...]
    mag = mag_ref[0]

    inb = (xt > minb) & (xt < maxb)
    u = (xt - ctr) * ibw
    q = u * u
    qp = jnp.where(inb, q, jnp.float32(_BIG))
    s = jnp.sum(qp, axis=1)
    y_ref[...] = (mag * jnp.exp(-s))[None]

    blk_any = jnp.max(jnp.where(s < _THRESH, 1.0, 0.0))

    @pl.when(pid == 0)
    def _():
        any_ref[...] = jnp.broadcast_to(blk_any, (1, 1))
        s0 = jnp.sum(q[0, :, 0])
        v0_ref[...] = jnp.broadcast_to(mag * jnp.exp(-s0), (1, 1))

    @pl.when(pid != 0)
    def _():
        any_ref[...] = jnp.maximum(any_ref[...], blk_any)


def _largest_divisor_leq(n, cap):
    for b in range(min(n, cap), 0, -1):
        if n % b == 0:
            return b
    return 1


def kernel(x, ctr, band_widths, mag):
    n, d = x.shape
    lanes = 128
    g = n // lanes
    blk = _largest_divisor_leq(g, 128)

    k = jnp.float32(_SUPPORT_K)
    minb = (ctr - k * band_widths).reshape(1, d, 1)
    maxb = (ctr + k * band_widths).reshape(1, d, 1)
    ctr3 = ctr.reshape(1, d, 1)
    ibw = (1.0 / band_widths).reshape(1, d, 1)
    xv = x.reshape(g, lanes, d)

    yv, any_f, v0 = pl.pallas_call(
        _bump_body,
        grid=(g // blk,),
        in_specs=[
            pl.BlockSpec((blk, lanes, d), lambda i: (i, 0, 0)),
            pl.BlockSpec((1, d, 1), lambda i: (0, 0, 0)),
            pl.BlockSpec((1, d, 1), lambda i: (0, 0, 0)),
            pl.BlockSpec((1, d, 1), lambda i: (0, 0, 0)),
            pl.BlockSpec((1, d, 1), lambda i: (0, 0, 0)),
            pl.BlockSpec(memory_space=pltpu.SMEM),
        ],
        out_specs=[
            pl.BlockSpec((1, blk, lanes), lambda i: (i, 0, 0)),
            pl.BlockSpec((1, 1), lambda i: (0, 0)),
            pl.BlockSpec((1, 1), lambda i: (0, 0)),
        ],
        out_shape=[
            jax.ShapeDtypeStruct((g // blk, blk, lanes), jnp.float32),
            jax.ShapeDtypeStruct((1, 1), jnp.float32),
            jax.ShapeDtypeStruct((1, 1), jnp.float32),
        ],
    )(xv, minb, maxb, ctr3, ibw, mag)

    y = yv.reshape(n)
    y0 = jnp.where(any_f[0, 0] > 0, y[0], v0[0, 0])
    return y.at[0].set(y0)
